# Initial kernel scaffold; baseline (speedup 1.0000x reference)
import jax
import jax.numpy as jnp
from jax.experimental import pallas as pl


def kernel(state, species_emb, holling_alpha_raw, W1, b1, W2, b2, W3, b3, Wq, bq, Wk, bk, r):
    raise NotImplementedError("write your pallas kernel here")


# speedup vs baseline: 9.4776x; 9.4776x over previous
import functools
import math

import jax
import jax.numpy as jnp
from jax.experimental import pallas as pl
from jax.experimental.pallas import tpu as pltpu

_B, _T, _N, _D, _H, _TOPK = 4, 64, 64, 24, 32, 8
_TB = 8


def _gelu(x):
    return 0.5 * x * (1.0 + jax.lax.erf(x * (1.0 / math.sqrt(2.0))))


def _fused_body(state_ref, emb_ref, alpha_ref, W1_ref, b1_ref, W2_ref, b2_ref,
                W3_ref, b3_ref, Wq_ref, bq_ref, Wk_ref, bk_ref, r_ref,
                lr_ref, attn_ref):
    X = state_ref[...]
    emb = emb_ref[...]
    alpha_raw = alpha_ref[...]
    W1 = W1_ref[...]
    b1 = b1_ref[...]
    W2 = W2_ref[...]
    b2 = b2_ref[...]
    W3 = W3_ref[...]
    b3 = b3_ref[...][None, :, :]
    Wq = Wq_ref[...]
    bq = bq_ref[...]
    Wk = Wk_ref[...]
    bk = bk_ref[...]
    rr = r_ref[...]

    f32 = jnp.float32
    TB, N, D, H = _TB, _N, _D, _H

    alpha = jax.nn.softplus(alpha_raw) + 0.01
    g = X / (1.0 + alpha * X)

    w_xi = W1[:, 0:1]
    w_xj = W1[:, 1:2]
    W_si = W1[:, 2:2 + D]
    W_sj = W1[:, 2 + D:2 + 2 * D]
    w_fl = W1[:, 50:51]
    w_lv = W1[:, 51:52]
    w_hl = W1[:, 52:53]
    w_hb = W1[:, 53:54]

    embSi = jax.lax.dot_general(emb, W_si, (((1,), (1,)), ((), ())),
                                preferred_element_type=f32)
    embSj = jax.lax.dot_general(emb, W_sj, (((1,), (1,)), ((), ())),
                                preferred_element_type=f32)

    C = X[:, :, None] * w_xi.T[None, :, :] + (embSi + b1)[None, :, :]
    A = X[:, :, None] * (w_xj + w_fl).T[None, :, :] + embSj[None, :, :] \
        + g[:, :, None] * w_hl.T[None, :, :]
    E = X[:, :, None] * w_lv.T[None, :, :] + g[:, :, None] * w_hb.T[None, :, :]

    h1 = C[:, :, None, :] + A[:, None, :, :] \
        + X[:, :, None, None] * E[:, None, :, :]
    h1 = _gelu(h1)
    h1f = h1.reshape(TB * N * N, H)
    h2 = _gelu(jax.lax.dot_general(h1f, W2, (((1,), (1,)), ((), ())),
                                   preferred_element_type=f32) + b2)
    h2r = h2.reshape(TB, N, N, H)
    msgs = jnp.sum(h2r * W3[None, None, :, :], axis=3) + b3

    feats = jnp.concatenate(
        [X[:, :, None], jnp.broadcast_to(emb[None, :, :], (TB, N, D))],
        axis=2)
    featsf = feats.reshape(TB * N, 1 + D)
    q = (jax.lax.dot_general(featsf, Wq, (((1,), (1,)), ((), ())))
         + bq).reshape(TB, N, D)
    k = (jax.lax.dot_general(featsf, Wk, (((1,), (1,)), ((), ())))
         + bk).reshape(TB, N, D)
    scores = jax.lax.dot_general(
        q, k, (((2,), (2,)), ((0,), (0,)))) / (D ** 0.5)

    NEG = jnp.float32(-3.0e38)
    jota = jax.lax.broadcasted_iota(jnp.int32, (TB, N, N), 2)
    work = scores
    sel = jnp.zeros((TB, N, N), dtype=jnp.bool_)
    for _ in range(_TOPK):
        m = jnp.max(work, axis=2, keepdims=True)
        is_m = work >= m
        fi = jnp.min(jnp.where(is_m, jota, N), axis=2, keepdims=True)
        pick = jota == fi
        sel = jnp.logical_or(sel, pick)
        work = jnp.where(pick, NEG, work)

    smax = jnp.max(jnp.where(sel, scores, NEG), axis=2, keepdims=True)
    p = jnp.where(sel, jnp.exp(scores - smax), 0.0)
    attn = p / jnp.sum(p, axis=2, keepdims=True)

    agg = jnp.sum(attn * msgs, axis=2)
    lr_ref[...] = rr + agg
    attn_ref[...] = attn


@functools.partial(jax.jit, static_argnames=())
def kernel(state, species_emb, holling_alpha_raw, W1, b1, W2, b2, W3, b3,
           Wq, bq, Wk, bk, r):
    B, T, N = state.shape
    R = B * T
    state2 = state.reshape(R, N)
    grid = (R // _TB,)

    def row_blk(i):
        return (i, 0)

    def rep2(i):
        return (0, 0)

    in_specs = [
        pl.BlockSpec((_TB, N), row_blk),
        pl.BlockSpec((N, _D), rep2),
        pl.BlockSpec((1, N), rep2),
        pl.BlockSpec((_H, 54), rep2),
        pl.BlockSpec((1, _H), rep2),
        pl.BlockSpec((_H, _H), rep2),
        pl.BlockSpec((1, _H), rep2),
        pl.BlockSpec((1, _H), rep2),
        pl.BlockSpec((1, N), rep2),
        pl.BlockSpec((_D, 1 + _D), rep2),
        pl.BlockSpec((1, _D), rep2),
        pl.BlockSpec((_D, 1 + _D), rep2),
        pl.BlockSpec((1, _D), rep2),
        pl.BlockSpec((1, N), rep2),
    ]
    out_specs = [
        pl.BlockSpec((_TB, N), row_blk),
        pl.BlockSpec((_TB, N, N), lambda i: (i, 0, 0)),
    ]
    out_shape = [
        jax.ShapeDtypeStruct((R, N), jnp.float32),
        jax.ShapeDtypeStruct((R, N, N), jnp.float32),
    ]
    lr, attn = pl.pallas_call(
        _fused_body,
        grid=grid,
        in_specs=in_specs,
        out_specs=out_specs,
        out_shape=out_shape,
    )(state2, species_emb, holling_alpha_raw[None, :], W1, b1[None, :], W2,
      b2[None, :], W3, jnp.broadcast_to(b3.reshape(1, 1), (1, N)), Wq,
      bq[None, :], Wk, bk[None, :], r[None, :])
    return lr.reshape(B, T, N), attn.reshape(B, T, N, N)


# speedup vs baseline: 13.6697x; 1.4423x over previous
import functools
import math

import jax
import jax.numpy as jnp
from jax.experimental import pallas as pl
from jax.experimental.pallas import tpu as pltpu

_B, _T, _N, _D, _H, _TOPK = 4, 64, 64, 24, 32, 8
_TB = 8


def _gelu(x):
    return 0.5 * x * (1.0 + jax.lax.erf(x * (1.0 / math.sqrt(2.0))))


def _fused_body(state_ref, emb_ref, alpha_ref, W1_ref, b1_ref, W2_ref, b2_ref,
                W3_ref, b3_ref, Wq_ref, bq_ref, Wk_ref, bk_ref, r_ref,
                lr_ref, attn_ref):
    X = state_ref[...]
    emb = emb_ref[...]
    alpha_raw = alpha_ref[...]
    W1 = W1_ref[...]
    b1 = b1_ref[...]
    W2 = W2_ref[...]
    b2 = b2_ref[...]
    W3b = W3_ref[...]
    b3 = b3_ref[...][None, :, :]
    Wq = Wq_ref[...]
    bq = bq_ref[...]
    Wk = Wk_ref[...]
    bk = bk_ref[...]
    rr = r_ref[...]

    f32 = jnp.float32
    TB, N, D, H = _TB, _N, _D, _H

    alpha = jax.nn.softplus(alpha_raw) + 0.01
    g = X / (1.0 + alpha * X)

    w_xi = W1[:, 0:1]
    w_xj = W1[:, 1:2]
    W_si = W1[:, 2:2 + D]
    W_sj = W1[:, 2 + D:2 + 2 * D]
    w_fl = W1[:, 50:51]
    w_lv = W1[:, 51:52]
    w_hl = W1[:, 52:53]
    w_hb = W1[:, 53:54]

    embSi = jax.lax.dot_general(W_si, emb, (((1,), (1,)), ((), ())),
                                preferred_element_type=f32)
    embSj = jax.lax.dot_general(W_sj, emb, (((1,), (1,)), ((), ())),
                                preferred_element_type=f32)

    Xb = X[None, :, :]
    gb = g[None, :, :]
    C = Xb * w_xi[:, :, None] + embSi[:, None, :] + b1[:, :, None]
    A = Xb * (w_xj + w_fl)[:, :, None] + embSj[:, None, :] \
        + gb * w_hl[:, :, None]
    E = Xb * w_lv[:, :, None] + gb * w_hb[:, :, None]

    h1 = C[:, :, :, None] + A[:, :, None, :] \
        + Xb[:, :, :, None] * E[:, :, None, :]
    h1 = _gelu(h1)
    h1f = h1.reshape(H, TB * N * N)
    h2 = _gelu(jax.lax.dot_general(W2, h1f, (((1,), (0,)), ((), ())),
                                   preferred_element_type=f32) + b2)
    h2r = h2.reshape(H, TB, N, N)
    acc = h2r[0] * W3b[0][None, None, :]
    for hh in range(1, H):
        acc = acc + h2r[hh] * W3b[hh][None, None, :]
    msgs = acc + b3

    feats = jnp.concatenate(
        [X[:, :, None], jnp.broadcast_to(emb[None, :, :], (TB, N, D))],
        axis=2)
    featsf = feats.reshape(TB * N, 1 + D)
    q = (jax.lax.dot_general(featsf, Wq, (((1,), (1,)), ((), ())))
         + bq).reshape(TB, N, D)
    k = (jax.lax.dot_general(featsf, Wk, (((1,), (1,)), ((), ())))
         + bk).reshape(TB, N, D)
    scores = jax.lax.dot_general(
        q, k, (((2,), (2,)), ((0,), (0,)))) / (D ** 0.5)

    NEG = jnp.float32(-3.0e38)
    jota = jax.lax.broadcasted_iota(jnp.int32, (TB, N, N), 2)
    work = scores
    sel = jnp.zeros((TB, N, N), dtype=jnp.bool_)
    for _ in range(_TOPK):
        m = jnp.max(work, axis=2, keepdims=True)
        is_m = work >= m
        fi = jnp.min(jnp.where(is_m, jota, N), axis=2, keepdims=True)
        pick = jota == fi
        sel = jnp.logical_or(sel, pick)
        work = jnp.where(pick, NEG, work)

    smax = jnp.max(jnp.where(sel, scores, NEG), axis=2, keepdims=True)
    p = jnp.where(sel, jnp.exp(scores - smax), 0.0)
    attn = p / jnp.sum(p, axis=2, keepdims=True)

    agg = jnp.sum(attn * msgs, axis=2)
    lr_ref[...] = rr + agg
    attn_ref[...] = attn


@functools.partial(jax.jit, static_argnames=())
def kernel(state, species_emb, holling_alpha_raw, W1, b1, W2, b2, W3, b3,
           Wq, bq, Wk, bk, r):
    B, T, N = state.shape
    R = B * T
    state2 = state.reshape(R, N)
    grid = (R // _TB,)

    def row_blk(i):
        return (i, 0)

    def rep2(i):
        return (0, 0)

    in_specs = [
        pl.BlockSpec((_TB, N), row_blk),
        pl.BlockSpec((N, _D), rep2),
        pl.BlockSpec((1, N), rep2),
        pl.BlockSpec((_H, 54), rep2),
        pl.BlockSpec((_H, 1), rep2),
        pl.BlockSpec((_H, _H), rep2),
        pl.BlockSpec((_H, 1), rep2),
        pl.BlockSpec((_H, N), rep2),
        pl.BlockSpec((1, N), rep2),
        pl.BlockSpec((_D, 1 + _D), rep2),
        pl.BlockSpec((1, _D), rep2),
        pl.BlockSpec((_D, 1 + _D), rep2),
        pl.BlockSpec((1, _D), rep2),
        pl.BlockSpec((1, N), rep2),
    ]
    out_specs = [
        pl.BlockSpec((_TB, N), row_blk),
        pl.BlockSpec((_TB, N, N), lambda i: (i, 0, 0)),
    ]
    out_shape = [
        jax.ShapeDtypeStruct((R, N), jnp.float32),
        jax.ShapeDtypeStruct((R, N, N), jnp.float32),
    ]
    lr, attn = pl.pallas_call(
        _fused_body,
        grid=grid,
        in_specs=in_specs,
        out_specs=out_specs,
        out_shape=out_shape,
    )(state2, species_emb, holling_alpha_raw[None, :], W1, b1[:, None], W2,
      b2[:, None], jnp.broadcast_to(W3.reshape(_H, 1), (_H, N)),
      jnp.broadcast_to(b3.reshape(1, 1), (1, N)), Wq,
      bq[None, :], Wk, bk[None, :], r[None, :])
    return lr.reshape(B, T, N), attn.reshape(B, T, N, N)


# speedup vs baseline: 29.1991x; 2.1360x over previous
import functools
import math

import jax
import jax.numpy as jnp
from jax import lax
from jax.experimental import pallas as pl
from jax.experimental.pallas import tpu as pltpu
from jax.experimental.pallas import tpu_sc as plsc

_B, _T, _N, _D, _H, _TOPK = 4, 64, 64, 24, 32, 8
_TB = 8
_R = _B * _T * _N
_NW = 32
_RPW = _R // _NW
_LANES = 16


def _gelu(x):
    return 0.5 * x * (1.0 + jax.lax.erf(x * (1.0 / math.sqrt(2.0))))


def _dense_body(state_ref, emb_ref, alpha_ref, W1_ref, b1_ref, W2_ref, b2_ref,
                W3_ref, b3_ref, Wq_ref, bq_ref, Wk_ref, bk_ref,
                svals_ref, sidx_ref, msel_ref):
    X = state_ref[...]
    emb = emb_ref[...]
    alpha_raw = alpha_ref[...]
    W1 = W1_ref[...]
    b1 = b1_ref[...]
    W2 = W2_ref[...]
    b2 = b2_ref[...]
    W3c = W3_ref[...]
    b3 = b3_ref[...]
    Wq = Wq_ref[...]
    bq = bq_ref[...]
    Wk = Wk_ref[...]
    bk = bk_ref[...]

    f32 = jnp.float32
    i32 = jnp.int32
    TB, N, D, H, K = _TB, _N, _D, _H, _TOPK
    S = N * K

    feats = jnp.concatenate(
        [X[:, :, None], jnp.broadcast_to(emb[None, :, :], (TB, N, D))],
        axis=2)
    featsf = feats.reshape(TB * N, 1 + D)
    q = (jax.lax.dot_general(featsf, Wq, (((1,), (1,)), ((), ())))
         + bq).reshape(TB, N, D)
    k = (jax.lax.dot_general(featsf, Wk, (((1,), (1,)), ((), ())))
         + bk).reshape(TB, N, D)
    scores = jax.lax.dot_general(
        q, k, (((2,), (2,)), ((0,), (0,)))) / (D ** 0.5)

    NEG = jnp.float32(-3.0e38)
    jota = jax.lax.broadcasted_iota(i32, (TB, N, N), 2)
    work = scores
    vals = []
    idxs = []
    for _r in range(K):
        m = jnp.max(work, axis=2)
        is_m = work >= m[:, :, None]
        fi = jnp.min(jnp.where(is_m, jota, N), axis=2)
        pick = jota == fi[:, :, None]
        work = jnp.where(pick, NEG, work)
        vals.append(m)
        idxs.append(fi)
    svals_ref[...] = jnp.stack(vals, axis=2).reshape(TB * N, K)
    sidx_ref[...] = jnp.stack(idxs, axis=2).reshape(TB * N, K)

    alpha = jax.nn.softplus(alpha_raw) + 0.01
    g = X / (1.0 + alpha * X)

    w_xi = W1[:, 0:1]
    w_xj = W1[:, 1:2]
    W_si = W1[:, 2:2 + D]
    W_sj = W1[:, 2 + D:2 + 2 * D]
    w_fl = W1[:, 50:51]
    w_lv = W1[:, 51:52]
    w_hl = W1[:, 52:53]
    w_hb = W1[:, 53:54]

    embSi = jax.lax.dot_general(W_si, emb, (((1,), (1,)), ((), ())),
                                preferred_element_type=f32)
    embSj = jax.lax.dot_general(W_sj, emb, (((1,), (1,)), ((), ())),
                                preferred_element_type=f32)

    Xb = X[None, :, :]
    gb = g[None, :, :]
    C = Xb * w_xi[:, :, None] + embSi[:, None, :] + b1[:, :, None]
    A = Xb * (w_xj + w_fl)[:, :, None] + embSj[:, None, :] \
        + gb * w_hl[:, :, None]
    E = Xb * w_lv[:, :, None] + gb * w_hb[:, :, None]

    tif = jnp.stack(idxs, axis=2).reshape(TB, S)
    jiota = jax.lax.broadcasted_iota(i32, (TB, N, S), 1)
    OH = jnp.where(tif[:, None, :] == jiota, 1.0, 0.0)
    riota = jax.lax.broadcasted_iota(i32, (N, S), 0)
    siota = jax.lax.broadcasted_iota(i32, (N, S), 1)
    OHC = jnp.where(lax.div(siota, K) == riota, 1.0, 0.0)
    Xrep = jax.lax.dot_general(X, OHC, (((1,), (0,)), ((), ())),
                               preferred_element_type=f32)
    OHX = OH * Xrep[:, None, :]

    h1_parts = []
    for t in range(TB):
        ct = jax.lax.dot_general(C[:, t, :], OHC, (((1,), (0,)), ((), ())),
                                 preferred_element_type=f32)
        at = jax.lax.dot_general(A[:, t, :], OH[t], (((1,), (0,)), ((), ())),
                                 preferred_element_type=f32)
        et = jax.lax.dot_general(E[:, t, :], OHX[t], (((1,), (0,)), ((), ())),
                                 preferred_element_type=f32)
        h1_parts.append(ct + at + et)
    h1s = jnp.concatenate(h1_parts, axis=1)
    h1s = _gelu(h1s)
    h2 = _gelu(jax.lax.dot_general(W2, h1s, (((1,), (0,)), ((), ())),
                                   preferred_element_type=f32) + b2)
    msel = jnp.sum(h2 * W3c, axis=0) + b3[0]
    msel_ref[...] = msel[None, None, :]


def _dense_call(state2, species_emb, holling_alpha_raw, W1, b1, W2, b2, W3,
                b3, Wq, bq, Wk, bk):
    N, K = _N, _TOPK
    RB = state2.shape[0]

    def row_blk(i):
        return (i, 0)

    def rep2(i):
        return (0, 0)

    in_specs = [
        pl.BlockSpec((_TB, N), row_blk),
        pl.BlockSpec((N, _D), rep2),
        pl.BlockSpec((1, N), rep2),
        pl.BlockSpec((_H, 54), rep2),
        pl.BlockSpec((_H, 1), rep2),
        pl.BlockSpec((_H, _H), rep2),
        pl.BlockSpec((_H, 1), rep2),
        pl.BlockSpec((_H, 1), rep2),
        pl.BlockSpec((1, _TB * N * K), rep2),
        pl.BlockSpec((_D, 1 + _D), rep2),
        pl.BlockSpec((1, _D), rep2),
        pl.BlockSpec((_D, 1 + _D), rep2),
        pl.BlockSpec((1, _D), rep2),
    ]
    out_specs = [
        pl.BlockSpec((_TB * N, K), row_blk),
        pl.BlockSpec((_TB * N, K), row_blk),
        pl.BlockSpec((1, 1, _TB * N * K), lambda i: (i, 0, 0)),
    ]
    out_shape = [
        jax.ShapeDtypeStruct((_R, K), jnp.float32),
        jax.ShapeDtypeStruct((_R, K), jnp.int32),
        jax.ShapeDtypeStruct((RB // _TB, 1, _TB * N * K), jnp.float32),
    ]
    return pl.pallas_call(
        _dense_body,
        grid=(RB // _TB,),
        in_specs=in_specs,
        out_specs=out_specs,
        out_shape=out_shape,
    )(state2, species_emb, holling_alpha_raw[None, :], W1, b1[:, None], W2,
      b2[:, None], W3.reshape(_H, 1),
      jnp.broadcast_to(b3.reshape(1, 1), (1, _TB * N * K)), Wq, bq[None, :],
      Wk, bk[None, :])


def _sc_attn_body(svals_hbm, sidx_hbm, msel_hbm, r_hbm, attn_hbm, lr_hbm,
                  sv_v, si_v, ms_v, r_v, at_v, lr_v):
    N, L, RPW, K = _N, _LANES, _RPW, _TOPK
    f32 = jnp.float32
    i32 = jnp.int32
    wid = lax.axis_index("s") * 2 + lax.axis_index("c")
    base = wid * RPW
    pltpu.sync_copy(svals_hbm.at[pl.ds(base * K, RPW * K)], sv_v)
    pltpu.sync_copy(sidx_hbm.at[pl.ds(base * K, RPW * K)], si_v)
    pltpu.sync_copy(msel_hbm.at[pl.ds(base * K, RPW * K)], ms_v)
    pltpu.sync_copy(r_hbm, r_v)
    iota = lax.iota(i32, L)

    @plsc.parallel_loop(0, RPW * N // L, unroll=8)
    def _(i):
        at_v[pl.ds(i * L, L)] = jnp.zeros((L,), f32)

    def chunk_body(cc, _):
        rbase = cc * L
        rowk = (rbase + iota) * K
        rowoff = (rbase + iota) * N
        rvec = r_v[pl.ds(lax.rem(rbase, N), L)]

        svs = [plsc.load_gather(sv_v, [rowk + kk]) for kk in range(K)]
        sis = [plsc.load_gather(si_v, [rowk + kk]) for kk in range(K)]
        mvs = [plsc.load_gather(ms_v, [rowk + kk]) for kk in range(K)]
        smax = svs[0]
        ps = [jnp.exp(v - smax) for v in svs]
        z = ps[0]
        for p in ps[1:]:
            z = z + p
        iz = 1.0 / z
        aggv = jnp.zeros((L,), f32)
        for kk in range(K):
            a = ps[kk] * iz
            plsc.store_scatter(at_v, [rowoff + sis[kk]], a)
            aggv = aggv + a * mvs[kk]
        lr_v[pl.ds(rbase, L)] = rvec + aggv
        return 0

    lax.fori_loop(0, RPW // L, chunk_body, 0)
    pltpu.sync_copy(at_v, attn_hbm.at[pl.ds(base * N, RPW * N)])
    pltpu.sync_copy(lr_v, lr_hbm.at[pl.ds(base, RPW)])


def _sc_attn_call(svals, sidx, msel, r):
    f32 = jnp.float32
    mesh = plsc.VectorSubcoreMesh(core_axis_name="c", subcore_axis_name="s")
    fn = functools.partial(
        pl.kernel,
        mesh=mesh,
        compiler_params=pltpu.CompilerParams(needs_layout_passes=False),
        out_type=[
            jax.ShapeDtypeStruct((_R * _N,), f32),
            jax.ShapeDtypeStruct((_R,), f32),
        ],
        scratch_types=[
            pltpu.VMEM((_RPW * _TOPK,), f32),
            pltpu.VMEM((_RPW * _TOPK,), jnp.int32),
            pltpu.VMEM((_RPW * _TOPK,), f32),
            pltpu.VMEM((_N,), f32),
            pltpu.VMEM((_RPW * _N,), f32),
            pltpu.VMEM((_RPW,), f32),
        ],
    )(_sc_attn_body)
    return fn(svals.reshape(_R * _TOPK), sidx.reshape(_R * _TOPK),
              msel.reshape(_R * _TOPK), r)


def kernel(state, species_emb, holling_alpha_raw, W1, b1, W2, b2, W3, b3,
           Wq, bq, Wk, bk, r):
    B, T, N = state.shape
    state2 = state.reshape(B * T, N)
    svals, sidx, msel = _dense_call(state2, species_emb, holling_alpha_raw,
                                    W1, b1, W2, b2, W3, b3, Wq, bq, Wk, bk)
    attn_flat, lr_flat = _sc_attn_call(svals, sidx, msel, r)
    return (lr_flat.reshape(B, T, N), attn_flat.reshape(B, T, N, N))


# speedup vs baseline: 33.6732x; 1.1532x over previous
import functools
import math

import jax
import jax.numpy as jnp
from jax import lax
from jax.experimental import pallas as pl
from jax.experimental.pallas import tpu as pltpu
from jax.experimental.pallas import tpu_sc as plsc

_B, _T, _N, _D, _H, _TOPK = 4, 64, 64, 24, 32, 8
_TB = 16
_R = _B * _T * _N
_NW = 32
_RPW = _R // _NW
_LANES = 16


def _gelu(x):
    return 0.5 * x * (1.0 + jax.lax.erf(x * (1.0 / math.sqrt(2.0))))


def _dense_body(state_ref, emb_ref, alpha_ref, W1_ref, b1_ref, W2_ref, b2_ref,
                W3_ref, b3_ref, Wq_ref, bq_ref, Wk_ref, bk_ref,
                svals_ref, sidx_ref, msel_ref):
    X = state_ref[...]
    emb = emb_ref[...]
    alpha_raw = alpha_ref[...]
    W1 = W1_ref[...]
    b1 = b1_ref[...]
    W2 = W2_ref[...]
    b2 = b2_ref[...]
    W3c = W3_ref[...]
    b3 = b3_ref[...]
    Wq = Wq_ref[...]
    bq = bq_ref[...]
    Wk = Wk_ref[...]
    bk = bk_ref[...]

    f32 = jnp.float32
    i32 = jnp.int32
    TB, N, D, H, K = _TB, _N, _D, _H, _TOPK
    S = N * K

    feats = jnp.concatenate(
        [X[:, :, None], jnp.broadcast_to(emb[None, :, :], (TB, N, D))],
        axis=2)
    featsf = feats.reshape(TB * N, 1 + D)
    q = (jax.lax.dot_general(featsf, Wq, (((1,), (1,)), ((), ())))
         + bq).reshape(TB, N, D)
    k = (jax.lax.dot_general(featsf, Wk, (((1,), (1,)), ((), ())))
         + bk).reshape(TB, N, D)
    scores = jax.lax.dot_general(
        q, k, (((2,), (2,)), ((0,), (0,)))) / (D ** 0.5)

    NEG = jnp.float32(-3.0e38)
    jota = jax.lax.broadcasted_iota(i32, (TB, N, N), 2)
    work = scores
    vals = []
    idxs = []
    for _r in range(K):
        m = jnp.max(work, axis=2)
        is_m = work >= m[:, :, None]
        fi = jnp.min(jnp.where(is_m, jota, N), axis=2)
        pick = jota == fi[:, :, None]
        work = jnp.where(pick, NEG, work)
        vals.append(m)
        idxs.append(fi)
    svals_ref[...] = jnp.stack(vals, axis=2).reshape(TB * N, K)
    sidx_ref[...] = jnp.stack(idxs, axis=2).reshape(TB * N, K)

    alpha = jax.nn.softplus(alpha_raw) + 0.01
    g = X / (1.0 + alpha * X)

    w_xi = W1[:, 0:1]
    w_xj = W1[:, 1:2]
    W_si = W1[:, 2:2 + D]
    W_sj = W1[:, 2 + D:2 + 2 * D]
    w_fl = W1[:, 50:51]
    w_lv = W1[:, 51:52]
    w_hl = W1[:, 52:53]
    w_hb = W1[:, 53:54]

    embSi = jax.lax.dot_general(W_si, emb, (((1,), (1,)), ((), ())),
                                preferred_element_type=f32)
    embSj = jax.lax.dot_general(W_sj, emb, (((1,), (1,)), ((), ())),
                                preferred_element_type=f32)

    Xb = X[None, :, :]
    gb = g[None, :, :]
    C = Xb * w_xi[:, :, None] + embSi[:, None, :] + b1[:, :, None]
    A = Xb * (w_xj + w_fl)[:, :, None] + embSj[:, None, :] \
        + gb * w_hl[:, :, None]
    E = Xb * w_lv[:, :, None] + gb * w_hb[:, :, None]

    tif = jnp.stack(idxs, axis=2).reshape(TB, S)
    jiota = jax.lax.broadcasted_iota(i32, (TB, N, S), 1)
    OH = jnp.where(tif[:, None, :] == jiota, 1.0, 0.0)
    riota = jax.lax.broadcasted_iota(i32, (N, S), 0)
    siota = jax.lax.broadcasted_iota(i32, (N, S), 1)
    OHC = jnp.where(lax.div(siota, K) == riota, 1.0, 0.0)
    Xrep = jax.lax.dot_general(X, OHC, (((1,), (0,)), ((), ())),
                               preferred_element_type=f32)
    OHX = OH * Xrep[:, None, :]

    h1_parts = []
    for t in range(TB):
        ct = jax.lax.dot_general(C[:, t, :], OHC, (((1,), (0,)), ((), ())),
                                 preferred_element_type=f32)
        at = jax.lax.dot_general(A[:, t, :], OH[t], (((1,), (0,)), ((), ())),
                                 preferred_element_type=f32)
        et = jax.lax.dot_general(E[:, t, :], OHX[t], (((1,), (0,)), ((), ())),
                                 preferred_element_type=f32)
        h1_parts.append(ct + at + et)
    h1s = jnp.concatenate(h1_parts, axis=1)
    h1s = _gelu(h1s)
    h2 = _gelu(jax.lax.dot_general(W2, h1s, (((1,), (0,)), ((), ())),
                                   preferred_element_type=f32) + b2)
    msel = jnp.sum(h2 * W3c, axis=0) + b3[0]
    msel_ref[...] = msel[None, None, :]


def _dense_call(state2, species_emb, holling_alpha_raw, W1, b1, W2, b2, W3,
                b3, Wq, bq, Wk, bk):
    N, K = _N, _TOPK
    RB = state2.shape[0]

    def row_blk(i):
        return (i, 0)

    def rep2(i):
        return (0, 0)

    in_specs = [
        pl.BlockSpec((_TB, N), row_blk),
        pl.BlockSpec((N, _D), rep2),
        pl.BlockSpec((1, N), rep2),
        pl.BlockSpec((_H, 54), rep2),
        pl.BlockSpec((_H, 1), rep2),
        pl.BlockSpec((_H, _H), rep2),
        pl.BlockSpec((_H, 1), rep2),
        pl.BlockSpec((_H, 1), rep2),
        pl.BlockSpec((1, _TB * N * K), rep2),
        pl.BlockSpec((_D, 1 + _D), rep2),
        pl.BlockSpec((1, _D), rep2),
        pl.BlockSpec((_D, 1 + _D), rep2),
        pl.BlockSpec((1, _D), rep2),
    ]
    out_specs = [
        pl.BlockSpec((_TB * N, K), row_blk),
        pl.BlockSpec((_TB * N, K), row_blk),
        pl.BlockSpec((1, 1, _TB * N * K), lambda i: (i, 0, 0)),
    ]
    out_shape = [
        jax.ShapeDtypeStruct((_R, K), jnp.float32),
        jax.ShapeDtypeStruct((_R, K), jnp.int32),
        jax.ShapeDtypeStruct((RB // _TB, 1, _TB * N * K), jnp.float32),
    ]
    return pl.pallas_call(
        _dense_body,
        grid=(RB // _TB,),
        in_specs=in_specs,
        out_specs=out_specs,
        out_shape=out_shape,
    )(state2, species_emb, holling_alpha_raw[None, :], W1, b1[:, None], W2,
      b2[:, None], W3.reshape(_H, 1),
      jnp.broadcast_to(b3.reshape(1, 1), (1, _TB * N * K)), Wq, bq[None, :],
      Wk, bk[None, :])


def _sc_attn_body(svals_hbm, sidx_hbm, msel_hbm, r_hbm, attn_hbm, lr_hbm,
                  sv_v, si_v, ms_v, r_v, at_v, lr_v):
    N, L, RPW, K = _N, _LANES, _RPW, _TOPK
    f32 = jnp.float32
    i32 = jnp.int32
    wid = lax.axis_index("s") * 2 + lax.axis_index("c")
    base = wid * RPW
    pltpu.sync_copy(svals_hbm.at[pl.ds(base * K, RPW * K)], sv_v)
    pltpu.sync_copy(sidx_hbm.at[pl.ds(base * K, RPW * K)], si_v)
    pltpu.sync_copy(msel_hbm.at[pl.ds(base * K, RPW * K)], ms_v)
    pltpu.sync_copy(r_hbm, r_v)
    iota = lax.iota(i32, L)

    @plsc.parallel_loop(0, RPW * N // L, unroll=8)
    def _(i):
        at_v[pl.ds(i * L, L)] = jnp.zeros((L,), f32)

    def chunk_body(cc, _):
        rbase = cc * L
        rowk = (rbase + iota) * K
        rowoff = (rbase + iota) * N
        rvec = r_v[pl.ds(lax.rem(rbase, N), L)]

        svs = [plsc.load_gather(sv_v, [rowk + kk]) for kk in range(K)]
        sis = [plsc.load_gather(si_v, [rowk + kk]) for kk in range(K)]
        mvs = [plsc.load_gather(ms_v, [rowk + kk]) for kk in range(K)]
        smax = svs[0]
        ps = [jnp.exp(v - smax) for v in svs]
        z = ps[0]
        for p in ps[1:]:
            z = z + p
        iz = 1.0 / z
        aggv = jnp.zeros((L,), f32)
        for kk in range(K):
            a = ps[kk] * iz
            plsc.store_scatter(at_v, [rowoff + sis[kk]], a)
            aggv = aggv + a * mvs[kk]
        lr_v[pl.ds(rbase, L)] = rvec + aggv
        return 0

    lax.fori_loop(0, RPW // L, chunk_body, 0)
    pltpu.sync_copy(at_v, attn_hbm.at[pl.ds(base * N, RPW * N)])
    pltpu.sync_copy(lr_v, lr_hbm.at[pl.ds(base, RPW)])


def _sc_attn_call(svals, sidx, msel, r):
    f32 = jnp.float32
    mesh = plsc.VectorSubcoreMesh(core_axis_name="c", subcore_axis_name="s")
    fn = functools.partial(
        pl.kernel,
        mesh=mesh,
        compiler_params=pltpu.CompilerParams(needs_layout_passes=False),
        out_type=[
            jax.ShapeDtypeStruct((_R * _N,), f32),
            jax.ShapeDtypeStruct((_R,), f32),
        ],
        scratch_types=[
            pltpu.VMEM((_RPW * _TOPK,), f32),
            pltpu.VMEM((_RPW * _TOPK,), jnp.int32),
            pltpu.VMEM((_RPW * _TOPK,), f32),
            pltpu.VMEM((_N,), f32),
            pltpu.VMEM((_RPW * _N,), f32),
            pltpu.VMEM((_RPW,), f32),
        ],
    )(_sc_attn_body)
    return fn(svals.reshape(_R * _TOPK), sidx.reshape(_R * _TOPK),
              msel.reshape(_R * _TOPK), r)


def kernel(state, species_emb, holling_alpha_raw, W1, b1, W2, b2, W3, b3,
           Wq, bq, Wk, bk, r):
    B, T, N = state.shape
    state2 = state.reshape(B * T, N)
    svals, sidx, msel = _dense_call(state2, species_emb, holling_alpha_raw,
                                    W1, b1, W2, b2, W3, b3, Wq, bq, Wk, bk)
    attn_flat, lr_flat = _sc_attn_call(svals, sidx, msel, r)
    return (lr_flat.reshape(B, T, N), attn_flat.reshape(B, T, N, N))
